# baseline (speedup 1.0000x reference)
import functools

import jax
import jax.numpy as jnp
from jax import lax
from jax.experimental import pallas as pl
from jax.experimental.pallas import tpu as pltpu

_EPS = float(jnp.float32(1e-07))
_DECAY = float(jnp.float32(0.99))
_COMMIT = float(jnp.float32(0.25))


def _vq_kernel(num_tiles, q_ref, emb_ref, ema_ref, cs_ref,
               lat_ref, loss_ref, idx_ref, nvq_ref, nema_ref, ncs_ref,
               dw_acc, cnt_acc, loss_acc):
    i = pl.program_id(0)
    tile, dim = q_ref.shape
    num_emb = emb_ref.shape[1]

    @pl.when(i == 0)
    def _init():
        dw_acc[...] = jnp.zeros_like(dw_acc)
        cnt_acc[...] = jnp.zeros_like(cnt_acc)
        loss_acc[...] = jnp.zeros_like(loss_acc)

    q = q_ref[...]
    emb = emb_ref[...]

    q2 = q * q
    s8 = q2[:, 0:8]
    for k in range(1, dim // 8):
        s8 = s8 + q2[:, 8 * k:8 * k + 8]
    s4 = s8[:, 0:4] + s8[:, 4:8]
    s2 = s4[:, 0:2] + s4[:, 2:4]
    qn = s2[:, 0:1] + s2[:, 1:2]

    en = jnp.sum(emb * emb, axis=0, keepdims=True)
    xe = jnp.dot(q.astype(jnp.bfloat16), emb.astype(jnp.bfloat16),
                 preferred_element_type=jnp.float32)
    dist = (qn - 2.0 * xe) + en

    chunk = num_emb // 4
    iota = lax.broadcasted_iota(jnp.int32, (tile, num_emb), 1)

    def chunk_minarg(c):
        dc = dist[:, c * chunk:(c + 1) * chunk]
        ic = iota[:, c * chunk:(c + 1) * chunk]
        mv = jnp.min(dc, axis=1, keepdims=True)
        mi = jnp.min(jnp.where(dc <= mv, ic, num_emb), axis=1,
                     keepdims=True)
        return mv, mi

    def bf16rt(x):
        return x.astype(jnp.bfloat16).astype(jnp.float32)

    acc_v, idx = chunk_minarg(0)
    acc_v = bf16rt(acc_v)
    for c in range(1, 4):
        mv, mi = chunk_minarg(c)
        better = mv < acc_v
        idx = jnp.where(better, mi, idx)
        acc_v = bf16rt(jnp.where(better, mv, acc_v))
    idx_ref[...] = idx

    onehot = (iota == idx).astype(jnp.bfloat16)
    quant = lax.dot_general(onehot, emb.astype(jnp.bfloat16),
                            (((1,), (1,)), ((), ())),
                            preferred_element_type=jnp.float32)
    lat_ref[...] = quant

    d = quant - q
    loss_acc[...] = loss_acc[...] + jnp.sum(d * d) / jnp.float32(dim)

    dw_acc[...] += lax.dot_general(q.astype(jnp.bfloat16), onehot,
                                   (((0,), (0,)), ((), ())),
                                   preferred_element_type=jnp.float32)
    cnt_acc[...] += jnp.sum(onehot.astype(jnp.float32), axis=0,
                            keepdims=True)

    @pl.when(i == num_tiles - 1)
    def _epilogue():
        one = jnp.float32(1.0)
        ncs = cs_ref[...] * _DECAY + cnt_acc[...] * (one - _DECAY)
        ncs_ref[...] = ncs
        nema = ema_ref[...] * _DECAY + dw_acc[...] * (one - _DECAY)
        nema_ref[...] = nema
        n = jnp.sum(ncs)
        upd = (ncs + _EPS) / (n + jnp.float32(num_emb) * _EPS) * n
        nvq_ref[...] = nema / upd
        loss_ref[...] = _COMMIT * loss_acc[...]


def kernel(query, vq_emb, ema_emb, cluster_sizes):
    dim = query.shape[-1]
    num_emb = vq_emb.shape[1]
    flat_q = query.reshape(-1, dim)
    tokens = flat_q.shape[0]
    tile = 1024
    num_tiles = tokens // tile
    cs2 = cluster_sizes.reshape(1, num_emb)

    out_shapes = (
        jax.ShapeDtypeStruct((tokens, dim), jnp.float32),
        jax.ShapeDtypeStruct((1, 1), jnp.float32),
        jax.ShapeDtypeStruct((tokens, 1), jnp.int32),
        jax.ShapeDtypeStruct((dim, num_emb), jnp.float32),
        jax.ShapeDtypeStruct((dim, num_emb), jnp.float32),
        jax.ShapeDtypeStruct((1, num_emb), jnp.float32),
    )

    grid = (num_tiles,)
    full = lambda i: (0, 0)
    lat, loss, idx, nvq, nema, ncs = pl.pallas_call(
        functools.partial(_vq_kernel, num_tiles),
        grid=grid,
        in_specs=[
            pl.BlockSpec((tile, dim), lambda i: (i, 0)),
            pl.BlockSpec((dim, num_emb), full),
            pl.BlockSpec((dim, num_emb), full),
            pl.BlockSpec((1, num_emb), full),
        ],
        out_specs=[
            pl.BlockSpec((tile, dim), lambda i: (i, 0)),
            pl.BlockSpec((1, 1), full),
            pl.BlockSpec((tile, 1), lambda i: (i, 0)),
            pl.BlockSpec((dim, num_emb), full),
            pl.BlockSpec((dim, num_emb), full),
            pl.BlockSpec((1, num_emb), full),
        ],
        out_shape=out_shapes,
        scratch_shapes=[
            pltpu.VMEM((dim, num_emb), jnp.float32),
            pltpu.VMEM((1, num_emb), jnp.float32),
            pltpu.VMEM((1, 1), jnp.float32),
        ],
        compiler_params=pltpu.CompilerParams(
            dimension_semantics=("arbitrary",),
        ),
    )(flat_q, vq_emb, ema_emb, cs2)

    latents = lat.reshape(query.shape)
    indices = idx.reshape(query.shape[:-1])
    return (latents, loss.reshape(()), indices, nvq, nema,
            ncs.reshape(num_emb))


# speedup vs baseline: 1.0057x; 1.0057x over previous
import functools

import jax
import jax.numpy as jnp
from jax import lax
from jax.experimental import pallas as pl
from jax.experimental.pallas import tpu as pltpu

_EPS = float(jnp.float32(1e-07))
_DECAY = float(jnp.float32(0.99))
_COMMIT = float(jnp.float32(0.25))


def _vq_kernel(num_tiles, q_ref, emb_ref, ema_ref, cs_ref,
               lat_ref, loss_ref, idx_ref, nvq_ref, nema_ref, ncs_ref,
               dw_acc, cnt_acc, loss_acc):
    i = pl.program_id(0)
    tile, dim = q_ref.shape
    num_emb = emb_ref.shape[1]

    @pl.when(i == 0)
    def _init():
        dw_acc[...] = jnp.zeros_like(dw_acc)
        cnt_acc[...] = jnp.zeros_like(cnt_acc)
        loss_acc[...] = jnp.zeros_like(loss_acc)

    q = q_ref[...]
    emb = emb_ref[...]

    q2 = q * q
    s8 = q2[:, 0:8]
    for k in range(1, dim // 8):
        s8 = s8 + q2[:, 8 * k:8 * k + 8]
    s4 = s8[:, 0:4] + s8[:, 4:8]
    s2 = s4[:, 0:2] + s4[:, 2:4]
    qn = s2[:, 0:1] + s2[:, 1:2]

    en = jnp.sum(emb * emb, axis=0, keepdims=True)
    xe = jnp.dot(q.astype(jnp.bfloat16), emb.astype(jnp.bfloat16),
                 preferred_element_type=jnp.float32)
    dist = (qn - 2.0 * xe) + en

    chunk = num_emb // 4
    iota = lax.broadcasted_iota(jnp.int32, (tile, num_emb), 1)

    def chunk_minarg(c):
        dc = dist[:, c * chunk:(c + 1) * chunk]
        ic = iota[:, c * chunk:(c + 1) * chunk]
        mv = jnp.min(dc, axis=1, keepdims=True)
        mi = jnp.min(jnp.where(dc <= mv, ic, num_emb), axis=1,
                     keepdims=True)
        return mv, mi

    def bf16rt(x):
        return x.astype(jnp.bfloat16).astype(jnp.float32)

    acc_v, idx = chunk_minarg(0)
    acc_v = bf16rt(acc_v)
    for c in range(1, 4):
        mv, mi = chunk_minarg(c)
        better = mv < acc_v
        idx = jnp.where(better, mi, idx)
        acc_v = bf16rt(jnp.where(better, mv, acc_v))
    idx_ref[...] = idx

    onehot = (iota == idx).astype(jnp.bfloat16)
    quant = lax.dot_general(onehot, emb.astype(jnp.bfloat16),
                            (((1,), (1,)), ((), ())),
                            preferred_element_type=jnp.float32)
    lat_ref[...] = quant

    d = quant - q
    loss_acc[...] = loss_acc[...] + jnp.sum(d * d) / jnp.float32(dim)

    dw_acc[...] += lax.dot_general(q.astype(jnp.bfloat16), onehot,
                                   (((0,), (0,)), ((), ())),
                                   preferred_element_type=jnp.float32)
    cnt_acc[...] += jnp.sum(onehot.astype(jnp.float32), axis=0,
                            keepdims=True)

    @pl.when(i == num_tiles - 1)
    def _epilogue():
        one = jnp.float32(1.0)
        ncs = cs_ref[...] * _DECAY + cnt_acc[...] * (one - _DECAY)
        ncs_ref[...] = ncs
        nema = ema_ref[...] * _DECAY + dw_acc[...] * (one - _DECAY)
        nema_ref[...] = nema
        n = jnp.sum(ncs)
        upd = (ncs + _EPS) / (n + jnp.float32(num_emb) * _EPS) * n
        nvq_ref[...] = nema / upd
        loss_ref[...] = _COMMIT * loss_acc[...]


def kernel(query, vq_emb, ema_emb, cluster_sizes):
    dim = query.shape[-1]
    num_emb = vq_emb.shape[1]
    flat_q = query.reshape(-1, dim)
    tokens = flat_q.shape[0]
    tile = 512
    num_tiles = tokens // tile
    cs2 = cluster_sizes.reshape(1, num_emb)

    out_shapes = (
        jax.ShapeDtypeStruct((tokens, dim), jnp.float32),
        jax.ShapeDtypeStruct((1, 1), jnp.float32),
        jax.ShapeDtypeStruct((tokens, 1), jnp.int32),
        jax.ShapeDtypeStruct((dim, num_emb), jnp.float32),
        jax.ShapeDtypeStruct((dim, num_emb), jnp.float32),
        jax.ShapeDtypeStruct((1, num_emb), jnp.float32),
    )

    grid = (num_tiles,)
    full = lambda i: (0, 0)
    lat, loss, idx, nvq, nema, ncs = pl.pallas_call(
        functools.partial(_vq_kernel, num_tiles),
        grid=grid,
        in_specs=[
            pl.BlockSpec((tile, dim), lambda i: (i, 0)),
            pl.BlockSpec((dim, num_emb), full),
            pl.BlockSpec((dim, num_emb), full),
            pl.BlockSpec((1, num_emb), full),
        ],
        out_specs=[
            pl.BlockSpec((tile, dim), lambda i: (i, 0)),
            pl.BlockSpec((1, 1), full),
            pl.BlockSpec((tile, 1), lambda i: (i, 0)),
            pl.BlockSpec((dim, num_emb), full),
            pl.BlockSpec((dim, num_emb), full),
            pl.BlockSpec((1, num_emb), full),
        ],
        out_shape=out_shapes,
        scratch_shapes=[
            pltpu.VMEM((dim, num_emb), jnp.float32),
            pltpu.VMEM((1, num_emb), jnp.float32),
            pltpu.VMEM((1, 1), jnp.float32),
        ],
        compiler_params=pltpu.CompilerParams(
            dimension_semantics=("arbitrary",),
        ),
    )(flat_q, vq_emb, ema_emb, cs2)

    latents = lat.reshape(query.shape)
    indices = idx.reshape(query.shape[:-1])
    return (latents, loss.reshape(()), indices, nvq, nema,
            ncs.reshape(num_emb))
